# Initial kernel scaffold; baseline (speedup 1.0000x reference)
import jax
import jax.numpy as jnp
from jax.experimental import pallas as pl


def kernel(grid_node_feats, params, mesh_node_feats, mesh_edge_feats, g2m_edge_feats, m2g_edge_feats, g2m_src_idx, g2m_dst_idx, m2m_src_idx, m2m_dst_idx, m2g_src_idx, m2g_dst_idx, per_variable_level_mean, per_variable_level_std):
    raise NotImplementedError("write your pallas kernel here")


# speedup vs baseline: 1.2292x; 1.2292x over previous
import functools

import jax
import jax.numpy as jnp
from jax import lax
from jax.experimental import pallas as pl
from jax.experimental.pallas import tpu as pltpu
from jax.experimental.pallas import tpu_sc as plsc

F32 = jnp.float32
I32 = jnp.int32
_PREC = lax.Precision.HIGHEST

_NC = 2
_NS = 16
_NW = _NC * _NS
_CH = 128
_BN = 512


def _rup(n, m):
    return (n + m - 1) // m * m


def _dot(a, b):
    return lax.dot_general(a, b, (((1,), (0,)), ((), ())),
                           precision=_PREC, preferred_element_type=F32)


def _z_from(x, w):
    if w.shape[0] <= 8:
        z = x[:, 0:1] * w[0:1, :]
        for k in range(1, w.shape[0]):
            z = z + x[:, k:k + 1] * w[k:k + 1, :]
        return z
    return _dot(x, w)


def _mlp_tc(mm, adds, b1, w2, b2, ln=None, res_first=False, final=None,
            bn=_BN):
    xs = [m[0] for m in mm]
    ws = [m[1] for m in mm]
    n = xs[0].shape[0]
    dout = w2.shape[1]
    n_mm, n_add = len(xs), len(adds)
    has_ln = ln is not None
    has_fin = final is not None

    arrays, specs = [], []

    def row(a):
        arrays.append(a)
        specs.append(pl.BlockSpec((bn, a.shape[1]), lambda i: (i, 0)))

    def full(a):
        a2 = a if a.ndim == 2 else a[None, :]
        arrays.append(a2)
        specs.append(pl.BlockSpec(a2.shape, lambda i: (0, 0)))

    for x in xs:
        row(x)
    for a in adds:
        row(a)
    if has_fin:
        row(final[2])
    for w in ws:
        full(w)
    full(b1)
    full(w2)
    full(b2)
    if has_ln:
        full(ln[0])
        full(ln[1])
    if has_fin:
        full(final[0])
        full(final[1])

    def body(*refs):
        p = 0
        xr = refs[p:p + n_mm]; p += n_mm
        ar = refs[p:p + n_add]; p += n_add
        if has_fin:
            xres_r = refs[p]; p += 1
        wr = refs[p:p + n_mm]; p += n_mm
        b1r = refs[p]; w2r = refs[p + 1]; b2r = refs[p + 2]; p += 3
        if has_ln:
            gr = refs[p]; ber = refs[p + 1]; p += 2
        if has_fin:
            scr = refs[p]; shr = refs[p + 1]; p += 2
        outr = refs[p]

        z = _z_from(xr[0][...], wr[0][...])
        for k in range(1, n_mm):
            z = z + _z_from(xr[k][...], wr[k][...])
        for a in ar:
            z = z + a[...]
        z = z + b1r[...]
        h = z / (1.0 + jnp.exp(-z))
        o = _dot(h, w2r[...]) + b2r[...]
        if has_ln:
            mu = jnp.mean(o, axis=-1, keepdims=True)
            var = jnp.mean((o - mu) ** 2, axis=-1, keepdims=True)
            o = (o - mu) * lax.rsqrt(var + 1e-5) * gr[...] + ber[...]
        if res_first:
            o = xr[0][...] + o
        if has_fin:
            o = o * scr[...] + shr[...] + xres_r[...]
        outr[...] = o

    return pl.pallas_call(
        body,
        grid=(n // bn,),
        in_specs=specs,
        out_specs=pl.BlockSpec((bn, dout), lambda i: (i, 0)),
        out_shape=jax.ShapeDtypeStruct((n, dout), F32),
    )(*arrays)


def _mlp_p(p, x, res_first=False):
    return _mlp_tc([(x, p['w1'])], [], p['b1'], p['w2'], p['b2'],
                   ln=(p['g'], p['be']) if 'g' in p else None,
                   res_first=res_first)


def _proj_tc(x, ws, bn=_BN):
    n = x.shape[0]
    m = len(ws)
    specs = [pl.BlockSpec((bn, x.shape[1]), lambda i: (i, 0))]
    for w in ws:
        specs.append(pl.BlockSpec(w.shape, lambda i: (0, 0)))

    def body(*refs):
        xv = refs[0][...]
        for w_r, o_r in zip(refs[1:1 + m], refs[1 + m:]):
            o_r[...] = _dot(xv, w_r[...])

    outs = pl.pallas_call(
        body,
        grid=(n // bn,),
        in_specs=specs,
        out_specs=[pl.BlockSpec((bn, w.shape[1]), lambda i: (i, 0))
                   for w in ws],
        out_shape=[jax.ShapeDtypeStruct((n, w.shape[1]), F32) for w in ws],
    )(x, *ws)
    return outs


def _sc_gather2(tab_s, tab_d, idx_s, idx_d):
    nch = idx_s.shape[1]
    e = _NW * nch * _CH
    mesh = plsc.VectorSubcoreMesh(core_axis_name="c", subcore_axis_name="s")
    out_t = (jax.ShapeDtypeStruct((e, 128), F32),
             jax.ShapeDtypeStruct((e, 128), F32))
    scratch = [pltpu.VMEM((nch, _CH), I32), pltpu.VMEM((nch, _CH), I32),
               pltpu.VMEM((_CH, 128), F32), pltpu.VMEM((_CH, 128), F32),
               pltpu.SemaphoreType.DMA, pltpu.SemaphoreType.DMA]

    @functools.partial(pl.kernel, out_type=out_t, mesh=mesh,
                       scratch_types=scratch)
    def k(ts, td, is_, id_, os_, od_, ia, ib, ra, rb, sa, sb):
        wid = lax.axis_index("s") * _NC + lax.axis_index("c")
        base = wid * (nch * _CH)
        pltpu.sync_copy(is_.at[wid], ia)
        pltpu.sync_copy(id_.at[wid], ib)

        def chunk(i, carry):
            off = base + i * _CH
            ca = pltpu.async_copy(ts.at[ia.at[i]], ra, sa)
            cb = pltpu.async_copy(td.at[ib.at[i]], rb, sb)
            ca.wait()
            pltpu.sync_copy(ra, os_.at[pl.ds(off, _CH)])
            cb.wait()
            pltpu.sync_copy(rb, od_.at[pl.ds(off, _CH)])
            return carry

        lax.fori_loop(0, nch, chunk, 0)

    return k(tab_s, tab_d, idx_s, idx_d)


def _sc_segsum(vals, idx, n_acc, w):
    nch = idx.shape[1]
    rpt = n_acc // _NS
    nfull, tail = rpt // _CH, rpt % _CH
    nzch = nfull + (1 if tail else 0)
    zeros = jnp.zeros((_CH, w), F32)
    rows = (jnp.arange(_NS, dtype=I32)[:, None] * rpt
            + jnp.clip(jnp.arange(nzch * _CH, dtype=I32), 0, rpt - 1)
            ).reshape(_NS, nzch, _CH)
    dv = vals.shape[1]
    mesh = plsc.VectorSubcoreMesh(core_axis_name="c", subcore_axis_name="s")
    scratch = [pltpu.VMEM((_CH, dv), F32), pltpu.VMEM((_CH, w), F32),
               pltpu.VMEM((nch, _CH), I32),
               pltpu.VMEM_SHARED((n_acc, w), F32),
               pltpu.SemaphoreType.DMA]

    @functools.partial(
        pl.kernel, out_type=jax.ShapeDtypeStruct((2, n_acc, w), F32),
        mesh=mesh, scratch_types=scratch)
    def k(vals_h, idx_h, z_h, out_h, vbuf, vbuf2, ibuf, acc, sem):
        c = lax.axis_index("c")
        s = lax.axis_index("s")
        wid = s * _NC + c
        r0 = s * rpt
        pltpu.sync_copy(z_h, vbuf2)
        pltpu.sync_copy(idx_h.at[wid], ibuf)

        def zero_batch(base_r, cnt):
            def zq(q, carry):
                pltpu.async_copy(vbuf2.at[0], acc.at[base_r + q], sem)
                return carry

            lax.fori_loop(0, cnt, zq, 0)
            if cnt == _CH:
                pltpu.make_async_copy(z_h, vbuf2, sem).wait()
            else:
                pltpu.make_async_copy(z_h.at[pl.ds(0, cnt)],
                                      vbuf2.at[pl.ds(0, cnt)], sem).wait()

        for j in range(nfull):
            zero_batch(r0 + j * _CH, _CH)
        if tail:
            zero_batch(r0 + nfull * _CH, tail)
        plsc.subcore_barrier()
        base = wid * (nch * _CH)

        def chunk(i, carry):
            off = base + i * _CH
            pltpu.sync_copy(vals_h.at[pl.ds(off, _CH)], vbuf)
            pltpu.sync_copy(vbuf, acc.at[ibuf.at[i]], add=True)
            return carry

        lax.fori_loop(0, nch, chunk, 0)
        plsc.subcore_barrier()

        def read_batch(base_r, cnt):
            def rq(q, carry):
                pltpu.async_copy(acc.at[base_r + q], vbuf2.at[q], sem)
                return carry

            lax.fori_loop(0, cnt, rq, 0)
            if cnt == _CH:
                pltpu.make_async_copy(z_h, vbuf2, sem).wait()
                pltpu.sync_copy(vbuf2, out_h.at[c, pl.ds(base_r, _CH)])
            else:
                pltpu.make_async_copy(z_h.at[pl.ds(0, cnt)],
                                      vbuf2.at[pl.ds(0, cnt)], sem).wait()
                pltpu.sync_copy(vbuf2.at[pl.ds(0, cnt)],
                                out_h.at[c, pl.ds(base_r, cnt)])

        for j in range(nfull):
            read_batch(r0 + j * _CH, _CH)
        if tail:
            read_batch(r0 + nfull * _CH, tail)

    return k(vals, idx, zeros)


_gather2 = _sc_gather2
_segsum = _sc_segsum


def _pad_rows(a, n):
    return jnp.pad(a, ((0, n - a.shape[0]),) + ((0, 0),) * (a.ndim - 1))


def _gather_idx(idx, e_pad):
    i = jnp.pad(idx.astype(I32), (0, e_pad - idx.shape[0]))
    return i.reshape(_NW, e_pad // (_NW * _CH), _CH)


def _scatter_idx(idx, e_pad, dummy):
    i = jnp.pad(idx.astype(I32), (0, e_pad - idx.shape[0]),
                constant_values=dummy)
    return i.reshape(_NW, e_pad // (_NW * _CH), _CH)


def _edge_block(p, e_lat, gs, gd):
    w1 = p['w1']
    return _mlp_tc([(e_lat, w1[:128])], [gs, gd], p['b1'], p['w2'], p['b2'],
                   ln=(p['g'], p['be']), res_first=True)


def _node_block(p, v, aggs):
    w1 = p['w1']
    mm = [(v, w1[:128])]
    for a, r0 in aggs:
        mm.append((a, w1[128 + r0:128 + r0 + a.shape[1]]))
    return _mlp_tc(mm, [], p['b1'], p['w2'], p['b2'],
                   ln=(p['g'], p['be']), res_first=True)


def kernel(grid_node_feats, params, mesh_node_feats, mesh_edge_feats,
           g2m_edge_feats, m2g_edge_feats, g2m_src_idx, g2m_dst_idx,
           m2m_src_idx, m2m_dst_idx, m2g_src_idx, m2g_dst_idx,
           per_variable_level_mean, per_variable_level_std):
    p = params
    x = grid_node_feats[0].astype(F32)
    ng, d_in = x.shape
    nm = mesh_node_feats.shape[0]
    d_out = per_variable_level_mean.shape[0]

    ngp = _rup(ng, _BN)
    nmp = _rup(nm, _BN)
    eg = _rup(g2m_edge_feats.shape[0], _NW * _CH)
    em_ = _rup(mesh_edge_feats.shape[0], _NW * _CH)
    emg = _rup(m2g_edge_feats.shape[0], _NW * _CH)
    nm_acc = _rup(nmp + 1, 128)

    xg = _pad_rows(x, ngp)

    vg = _mlp_p(p['enc_vg'], xg)
    vm = _mlp_p(p['enc_vm'], _pad_rows(mesh_node_feats, nmp))
    em = _mlp_p(p['enc_em'], _pad_rows(mesh_edge_feats, em_))
    eg2m = _mlp_p(p['enc_eg2m'], _pad_rows(g2m_edge_feats, eg))
    em2g = _mlp_p(p['enc_em2g'], _pad_rows(m2g_edge_feats, emg))

    g2m_s_g = _gather_idx(g2m_src_idx, eg)
    g2m_d_g = _gather_idx(g2m_dst_idx, eg)
    g2m_d_s = _scatter_idx(g2m_dst_idx, eg, nmp)
    m2m_s_g = _gather_idx(m2m_src_idx, em_)
    m2m_d_g = _gather_idx(m2m_dst_idx, em_)
    m2m_d_s = _scatter_idx(m2m_dst_idx, em_, nmp)
    m2g_s_g = _gather_idx(m2g_src_idx, emg)
    m2g_d_g = _gather_idx(m2g_dst_idx, emg)

    w1 = p['g2m_edge']['w1']
    (ts,) = _proj_tc(vg, [w1[128:256]])
    (td,) = _proj_tc(vm, [w1[256:384]])
    gs, gd = _gather2(ts, td, g2m_s_g, g2m_d_g)
    eg2m = _edge_block(p['g2m_edge'], eg2m, gs, gd)
    parts = _segsum(eg2m, g2m_d_s, nm_acc, 128)
    vm = _node_block(p['g2m_node'], vm,
                     [(parts[0, :nmp], 0), (parts[1, :nmp], 0)])
    vg = _mlp_p(p['g2m_grid'], vg, res_first=True)

    for sp in p['proc']:
        w1 = sp['edge']['w1']
        ts, td = _proj_tc(vm, [w1[128:256], w1[256:384]])
        gs, gd = _gather2(ts, td, m2m_s_g, m2m_d_g)
        em = _edge_block(sp['edge'], em, gs, gd)
        parts = _segsum(em, m2m_d_s, nm_acc, 128)
        vm = _node_block(sp['node'], vm,
                         [(parts[0, :nmp], 0), (parts[1, :nmp], 0)])

    w1 = p['dec_edge']['w1']
    (ts,) = _proj_tc(vm, [w1[128:256]])
    (td,) = _proj_tc(vg, [w1[256:384]])
    gs, gd = _gather2(ts, td, m2g_s_g, m2g_d_g)
    em2g = _edge_block(p['dec_edge'], em2g, gs, gd)
    rr = 8448
    n_acc_p = _rup(rr + 1, 128)
    nph = -(-ngp // rr)
    p0s, p1s = [], []
    for ph in range(nph):
        lo = ph * rr
        mid = jnp.asarray(m2g_dst_idx, dtype=I32)
        idx_p = jnp.where((mid >= lo) & (mid < lo + rr), mid - lo, rr)
        idx_p = _scatter_idx(idx_p, emg, rr)
        parts = _segsum(em2g, idx_p, n_acc_p, 128)
        p0s.append(parts[0, :rr])
        p1s.append(parts[1, :rr])
    agg0 = jnp.concatenate(p0s, axis=0)[:ngp]
    agg1 = jnp.concatenate(p1s, axis=0)[:ngp]
    vg = _node_block(p['dec_node'], vg, [(agg0, 0), (agg1, 0)])

    fp = p['dec_final']
    xres = xg[:, d_out:2 * d_out]
    out = _mlp_tc([(vg, fp['w1'])], [], fp['b1'], fp['w2'], fp['b2'],
                  ln=None, final=(per_variable_level_std,
                                  per_variable_level_mean, xres))
    return out[:ng][None]


# speedup vs baseline: 1.2361x; 1.0056x over previous
import functools

import jax
import jax.numpy as jnp
from jax import lax
from jax.experimental import pallas as pl
from jax.experimental.pallas import tpu as pltpu
from jax.experimental.pallas import tpu_sc as plsc

F32 = jnp.float32
I32 = jnp.int32
_PREC = lax.Precision.HIGHEST

_NC = 2
_NS = 16
_NW = _NC * _NS
_CH = 128
_BN = 512


def _rup(n, m):
    return (n + m - 1) // m * m


def _dot(a, b):
    return lax.dot_general(a, b, (((1,), (0,)), ((), ())),
                           precision=_PREC, preferred_element_type=F32)


def _z_from(x, w):
    if w.shape[0] <= 8:
        z = x[:, 0:1] * w[0:1, :]
        for k in range(1, w.shape[0]):
            z = z + x[:, k:k + 1] * w[k:k + 1, :]
        return z
    return _dot(x, w)


def _mlp_tc(mm, adds, b1, w2, b2, ln=None, res_first=False, final=None,
            bn=_BN):
    xs = [m[0] for m in mm]
    ws = [m[1] for m in mm]
    n = xs[0].shape[0]
    dout = w2.shape[1]
    n_mm, n_add = len(xs), len(adds)
    has_ln = ln is not None
    has_fin = final is not None

    arrays, specs = [], []

    def row(a):
        arrays.append(a)
        specs.append(pl.BlockSpec((bn, a.shape[1]), lambda i: (i, 0)))

    def full(a):
        a2 = a if a.ndim == 2 else a[None, :]
        arrays.append(a2)
        specs.append(pl.BlockSpec(a2.shape, lambda i: (0, 0)))

    for x in xs:
        row(x)
    for a in adds:
        row(a)
    if has_fin:
        row(final[2])
    for w in ws:
        full(w)
    full(b1)
    full(w2)
    full(b2)
    if has_ln:
        full(ln[0])
        full(ln[1])
    if has_fin:
        full(final[0])
        full(final[1])

    def body(*refs):
        p = 0
        xr = refs[p:p + n_mm]; p += n_mm
        ar = refs[p:p + n_add]; p += n_add
        if has_fin:
            xres_r = refs[p]; p += 1
        wr = refs[p:p + n_mm]; p += n_mm
        b1r = refs[p]; w2r = refs[p + 1]; b2r = refs[p + 2]; p += 3
        if has_ln:
            gr = refs[p]; ber = refs[p + 1]; p += 2
        if has_fin:
            scr = refs[p]; shr = refs[p + 1]; p += 2
        outr = refs[p]

        z = _z_from(xr[0][...], wr[0][...])
        for k in range(1, n_mm):
            z = z + _z_from(xr[k][...], wr[k][...])
        for a in ar:
            z = z + a[...]
        z = z + b1r[...]
        h = z / (1.0 + jnp.exp(-z))
        o = _dot(h, w2r[...]) + b2r[...]
        if has_ln:
            mu = jnp.mean(o, axis=-1, keepdims=True)
            var = jnp.mean((o - mu) ** 2, axis=-1, keepdims=True)
            o = (o - mu) * lax.rsqrt(var + 1e-5) * gr[...] + ber[...]
        if res_first:
            o = xr[0][...] + o
        if has_fin:
            o = o * scr[...] + shr[...] + xres_r[...]
        outr[...] = o

    return pl.pallas_call(
        body,
        grid=(n // bn,),
        in_specs=specs,
        out_specs=pl.BlockSpec((bn, dout), lambda i: (i, 0)),
        out_shape=jax.ShapeDtypeStruct((n, dout), F32),
    )(*arrays)


def _mlp_p(p, x, res_first=False):
    return _mlp_tc([(x, p['w1'])], [], p['b1'], p['w2'], p['b2'],
                   ln=(p['g'], p['be']) if 'g' in p else None,
                   res_first=res_first)


def _proj_tc(x, ws, bn=_BN):
    n = x.shape[0]
    m = len(ws)
    specs = [pl.BlockSpec((bn, x.shape[1]), lambda i: (i, 0))]
    for w in ws:
        specs.append(pl.BlockSpec(w.shape, lambda i: (0, 0)))

    def body(*refs):
        xv = refs[0][...]
        for w_r, o_r in zip(refs[1:1 + m], refs[1 + m:]):
            o_r[...] = _dot(xv, w_r[...])

    outs = pl.pallas_call(
        body,
        grid=(n // bn,),
        in_specs=specs,
        out_specs=[pl.BlockSpec((bn, w.shape[1]), lambda i: (i, 0))
                   for w in ws],
        out_shape=[jax.ShapeDtypeStruct((n, w.shape[1]), F32) for w in ws],
    )(x, *ws)
    return outs


def _sc_gather2(tab_s, tab_d, idx_s, idx_d):
    nch = idx_s.shape[1]
    e = _NW * nch * _CH
    npair, gtail = nch // 2, nch % 2
    mesh = plsc.VectorSubcoreMesh(core_axis_name="c", subcore_axis_name="s")
    out_t = (jax.ShapeDtypeStruct((e, 128), F32),
             jax.ShapeDtypeStruct((e, 128), F32))
    scratch = [pltpu.VMEM((nch, _CH), I32), pltpu.VMEM((nch, _CH), I32),
               pltpu.VMEM((_CH, 128), F32), pltpu.VMEM((_CH, 128), F32),
               pltpu.VMEM((_CH, 128), F32), pltpu.VMEM((_CH, 128), F32),
               pltpu.SemaphoreType.DMA, pltpu.SemaphoreType.DMA,
               pltpu.SemaphoreType.DMA, pltpu.SemaphoreType.DMA,
               pltpu.SemaphoreType.DMA]

    @functools.partial(pl.kernel, out_type=out_t, mesh=mesh,
                       scratch_types=scratch)
    def k(ts, td, is_, id_, os_, od_, ia, ib, ra0, rb0, ra1, rb1,
          sa0, sb0, sa1, sb1, swb):
        wid = lax.axis_index("s") * _NC + lax.axis_index("c")
        base = wid * (nch * _CH)
        pltpu.sync_copy(is_.at[wid], ia)
        pltpu.sync_copy(id_.at[wid], ib)

        def pair(t, carry):
            i0 = 2 * t
            off0 = base + i0 * _CH
            off1 = off0 + _CH

            @pl.when(t > 0)
            def _():
                for r in (ra0, rb0, ra1, rb1):
                    pltpu.make_async_copy(r, os_.at[pl.ds(0, _CH)],
                                          swb).wait()

            ca0 = pltpu.async_copy(ts.at[ia.at[i0]], ra0, sa0)
            cb0 = pltpu.async_copy(td.at[ib.at[i0]], rb0, sb0)
            ca1 = pltpu.async_copy(ts.at[ia.at[i0 + 1]], ra1, sa1)
            cb1 = pltpu.async_copy(td.at[ib.at[i0 + 1]], rb1, sb1)
            ca0.wait()
            pltpu.async_copy(ra0, os_.at[pl.ds(off0, _CH)], swb)
            cb0.wait()
            pltpu.async_copy(rb0, od_.at[pl.ds(off0, _CH)], swb)
            ca1.wait()
            pltpu.async_copy(ra1, os_.at[pl.ds(off1, _CH)], swb)
            cb1.wait()
            pltpu.async_copy(rb1, od_.at[pl.ds(off1, _CH)], swb)
            return carry

        lax.fori_loop(0, npair, pair, 0)
        if npair:
            for r in (ra0, rb0, ra1, rb1):
                pltpu.make_async_copy(r, os_.at[pl.ds(0, _CH)], swb).wait()
        if gtail:
            i0 = nch - 1
            off = base + i0 * _CH
            ca = pltpu.async_copy(ts.at[ia.at[i0]], ra0, sa0)
            cb = pltpu.async_copy(td.at[ib.at[i0]], rb0, sb0)
            ca.wait()
            pltpu.sync_copy(ra0, os_.at[pl.ds(off, _CH)])
            cb.wait()
            pltpu.sync_copy(rb0, od_.at[pl.ds(off, _CH)])

    return k(tab_s, tab_d, idx_s, idx_d)


def _sc_segsum(vals, idx, n_acc, w):
    nch = idx.shape[1]
    rpt = n_acc // _NS
    nfull, tail = rpt // _CH, rpt % _CH
    nzch = nfull + (1 if tail else 0)
    zeros = jnp.zeros((_CH, w), F32)
    rows = (jnp.arange(_NS, dtype=I32)[:, None] * rpt
            + jnp.clip(jnp.arange(nzch * _CH, dtype=I32), 0, rpt - 1)
            ).reshape(_NS, nzch, _CH)
    dv = vals.shape[1]
    mesh = plsc.VectorSubcoreMesh(core_axis_name="c", subcore_axis_name="s")
    scratch = [pltpu.VMEM((_CH, dv), F32), pltpu.VMEM((_CH, w), F32),
               pltpu.VMEM((nch, _CH), I32),
               pltpu.VMEM_SHARED((n_acc, w), F32),
               pltpu.SemaphoreType.DMA]

    @functools.partial(
        pl.kernel, out_type=jax.ShapeDtypeStruct((2, n_acc, w), F32),
        mesh=mesh, scratch_types=scratch)
    def k(vals_h, idx_h, z_h, out_h, vbuf, vbuf2, ibuf, acc, sem):
        c = lax.axis_index("c")
        s = lax.axis_index("s")
        wid = s * _NC + c
        r0 = s * rpt
        pltpu.sync_copy(z_h, vbuf2)
        pltpu.sync_copy(idx_h.at[wid], ibuf)

        def zero_batch(base_r, cnt):
            def zq(q, carry):
                pltpu.async_copy(vbuf2.at[0], acc.at[base_r + q], sem)
                return carry

            lax.fori_loop(0, cnt, zq, 0)
            if cnt == _CH:
                pltpu.make_async_copy(z_h, vbuf2, sem).wait()
            else:
                pltpu.make_async_copy(z_h.at[pl.ds(0, cnt)],
                                      vbuf2.at[pl.ds(0, cnt)], sem).wait()

        for j in range(nfull):
            zero_batch(r0 + j * _CH, _CH)
        if tail:
            zero_batch(r0 + nfull * _CH, tail)
        plsc.subcore_barrier()
        base = wid * (nch * _CH)

        def chunk(i, carry):
            off = base + i * _CH
            pltpu.sync_copy(vals_h.at[pl.ds(off, _CH)], vbuf)
            pltpu.sync_copy(vbuf, acc.at[ibuf.at[i]], add=True)
            return carry

        lax.fori_loop(0, nch, chunk, 0)
        plsc.subcore_barrier()

        def read_batch(base_r, cnt):
            def rq(q, carry):
                pltpu.async_copy(acc.at[base_r + q], vbuf2.at[q], sem)
                return carry

            lax.fori_loop(0, cnt, rq, 0)
            if cnt == _CH:
                pltpu.make_async_copy(z_h, vbuf2, sem).wait()
                pltpu.sync_copy(vbuf2, out_h.at[c, pl.ds(base_r, _CH)])
            else:
                pltpu.make_async_copy(z_h.at[pl.ds(0, cnt)],
                                      vbuf2.at[pl.ds(0, cnt)], sem).wait()
                pltpu.sync_copy(vbuf2.at[pl.ds(0, cnt)],
                                out_h.at[c, pl.ds(base_r, cnt)])

        for j in range(nfull):
            read_batch(r0 + j * _CH, _CH)
        if tail:
            read_batch(r0 + nfull * _CH, tail)

    return k(vals, idx, zeros)


_gather2 = _sc_gather2
_segsum = _sc_segsum


def _pad_rows(a, n):
    return jnp.pad(a, ((0, n - a.shape[0]),) + ((0, 0),) * (a.ndim - 1))


def _gather_idx(idx, e_pad):
    i = jnp.pad(idx.astype(I32), (0, e_pad - idx.shape[0]))
    return i.reshape(_NW, e_pad // (_NW * _CH), _CH)


def _scatter_idx(idx, e_pad, dummy):
    i = jnp.pad(idx.astype(I32), (0, e_pad - idx.shape[0]),
                constant_values=dummy)
    return i.reshape(_NW, e_pad // (_NW * _CH), _CH)


def _edge_block(p, e_lat, gs, gd):
    w1 = p['w1']
    return _mlp_tc([(e_lat, w1[:128])], [gs, gd], p['b1'], p['w2'], p['b2'],
                   ln=(p['g'], p['be']), res_first=True)


def _node_block(p, v, aggs):
    w1 = p['w1']
    mm = [(v, w1[:128])]
    for a, r0 in aggs:
        mm.append((a, w1[128 + r0:128 + r0 + a.shape[1]]))
    return _mlp_tc(mm, [], p['b1'], p['w2'], p['b2'],
                   ln=(p['g'], p['be']), res_first=True)


def kernel(grid_node_feats, params, mesh_node_feats, mesh_edge_feats,
           g2m_edge_feats, m2g_edge_feats, g2m_src_idx, g2m_dst_idx,
           m2m_src_idx, m2m_dst_idx, m2g_src_idx, m2g_dst_idx,
           per_variable_level_mean, per_variable_level_std):
    p = params
    x = grid_node_feats[0].astype(F32)
    ng, d_in = x.shape
    nm = mesh_node_feats.shape[0]
    d_out = per_variable_level_mean.shape[0]

    ngp = _rup(ng, _BN)
    nmp = _rup(nm, _BN)
    eg = _rup(g2m_edge_feats.shape[0], _NW * _CH)
    em_ = _rup(mesh_edge_feats.shape[0], _NW * _CH)
    emg = _rup(m2g_edge_feats.shape[0], _NW * _CH)
    nm_acc = _rup(nmp + 1, 128)

    xg = _pad_rows(x, ngp)

    vg = _mlp_p(p['enc_vg'], xg)
    vm = _mlp_p(p['enc_vm'], _pad_rows(mesh_node_feats, nmp))
    em = _mlp_p(p['enc_em'], _pad_rows(mesh_edge_feats, em_))
    eg2m = _mlp_p(p['enc_eg2m'], _pad_rows(g2m_edge_feats, eg))
    em2g = _mlp_p(p['enc_em2g'], _pad_rows(m2g_edge_feats, emg))

    g2m_s_g = _gather_idx(g2m_src_idx, eg)
    g2m_d_g = _gather_idx(g2m_dst_idx, eg)
    g2m_d_s = _scatter_idx(g2m_dst_idx, eg, nmp)
    m2m_s_g = _gather_idx(m2m_src_idx, em_)
    m2m_d_g = _gather_idx(m2m_dst_idx, em_)
    m2m_d_s = _scatter_idx(m2m_dst_idx, em_, nmp)
    m2g_s_g = _gather_idx(m2g_src_idx, emg)
    m2g_d_g = _gather_idx(m2g_dst_idx, emg)

    w1 = p['g2m_edge']['w1']
    (ts,) = _proj_tc(vg, [w1[128:256]])
    (td,) = _proj_tc(vm, [w1[256:384]])
    gs, gd = _gather2(ts, td, g2m_s_g, g2m_d_g)
    eg2m = _edge_block(p['g2m_edge'], eg2m, gs, gd)
    parts = _segsum(eg2m, g2m_d_s, nm_acc, 128)
    vm = _node_block(p['g2m_node'], vm,
                     [(parts[0, :nmp], 0), (parts[1, :nmp], 0)])
    vg = _mlp_p(p['g2m_grid'], vg, res_first=True)

    for sp in p['proc']:
        w1 = sp['edge']['w1']
        ts, td = _proj_tc(vm, [w1[128:256], w1[256:384]])
        gs, gd = _gather2(ts, td, m2m_s_g, m2m_d_g)
        em = _edge_block(sp['edge'], em, gs, gd)
        parts = _segsum(em, m2m_d_s, nm_acc, 128)
        vm = _node_block(sp['node'], vm,
                         [(parts[0, :nmp], 0), (parts[1, :nmp], 0)])

    w1 = p['dec_edge']['w1']
    (ts,) = _proj_tc(vm, [w1[128:256]])
    (td,) = _proj_tc(vg, [w1[256:384]])
    gs, gd = _gather2(ts, td, m2g_s_g, m2g_d_g)
    em2g = _edge_block(p['dec_edge'], em2g, gs, gd)
    rr = 8448
    n_acc_p = _rup(rr + 1, 128)
    nph = -(-ngp // rr)
    p0s, p1s = [], []
    for ph in range(nph):
        lo = ph * rr
        mid = jnp.asarray(m2g_dst_idx, dtype=I32)
        idx_p = jnp.where((mid >= lo) & (mid < lo + rr), mid - lo, rr)
        idx_p = _scatter_idx(idx_p, emg, rr)
        parts = _segsum(em2g, idx_p, n_acc_p, 128)
        p0s.append(parts[0, :rr])
        p1s.append(parts[1, :rr])
    agg0 = jnp.concatenate(p0s, axis=0)[:ngp]
    agg1 = jnp.concatenate(p1s, axis=0)[:ngp]
    vg = _node_block(p['dec_node'], vg, [(agg0, 0), (agg1, 0)])

    fp = p['dec_final']
    xres = xg[:, d_out:2 * d_out]
    out = _mlp_tc([(vg, fp['w1'])], [], fp['b1'], fp['w2'], fp['b2'],
                  ln=None, final=(per_variable_level_std,
                                  per_variable_level_mean, xres))
    return out[:ng][None]
